# baseline (speedup 1.0000x reference)
import functools

import jax
import jax.numpy as jnp
from jax import lax
from jax.experimental import pallas as pl
from jax.experimental.pallas import tpu as pltpu
from jax.experimental.pallas import tpu_sc as plsc

DEPTH = 1000
ON_VALUE = 1.0
OFF_VALUE = 0.0

NUM_CORES = 2
NUM_SUBCORES = 16
NUM_WORKERS = NUM_CORES * NUM_SUBCORES
LANES = 16

CHUNK_B = 2
F_PAD = 32
D_PAD = 1024
BATCH_WORDS = F_PAD * D_PAD

SC_BATCHES = 256
TC_BLOCK_B = 16


def _one_hot_sc_image(idx_flat, f_total):
  batches_per_worker = SC_BATCHES // NUM_WORKERS
  n_chunks = batches_per_worker // CHUNK_B
  chunk_rows = CHUNK_B * f_total
  rows_per_worker = batches_per_worker * f_total
  n_groups = -(-chunk_rows // LANES)

  mesh = plsc.VectorSubcoreMesh(core_axis_name="c", subcore_axis_name="s")

  @functools.partial(
      pl.kernel,
      mesh=mesh,
      out_type=jax.ShapeDtypeStruct((SC_BATCHES * BATCH_WORDS,), jnp.float32),
      scratch_types=[
          pltpu.VMEM((rows_per_worker,), jnp.int32),
          pltpu.VMEM((CHUNK_B * BATCH_WORDS,), jnp.float32),
      ],
      compiler_params=pltpu.CompilerParams(needs_layout_passes=False),
  )
  def k(idx_hbm, out_hbm, idx_v, buf):
    wid = lax.axis_index("s") * NUM_CORES + lax.axis_index("c")
    batch0 = wid * batches_per_worker

    pltpu.sync_copy(idx_hbm.at[pl.ds(batch0 * f_total, rows_per_worker)],
                    idx_v)

    zeros16 = jnp.zeros((LANES,), jnp.float32)

    def zero_body(i, _):
      base = i * (8 * LANES)
      for u in range(8):
        buf[pl.ds(base + u * LANES, LANES)] = zeros16
      return 0

    lax.fori_loop(0, CHUNK_B * BATCH_WORDS // (8 * LANES), zero_body, 0)

    lane = lax.iota(jnp.int32, LANES)
    ones16 = jnp.full((LANES,), jnp.float32(ON_VALUE))

    def scatter_chunk(c, val16):
      for g in range(n_groups):
        j = lane + g * LANES
        mask = j < chunk_rows if (g + 1) * LANES > chunk_rows else None
        d = plsc.load_gather(idx_v, [j + c * chunk_rows], mask=mask)
        b = jnp.where(j >= f_total, 1, 0)
        f = j - b * f_total
        off = (b * BATCH_WORDS + (f >> 3) * (8 * D_PAD) + (d >> 7) * 1024
               + (f & 7) * 128 + (d & 127))
        plsc.store_scatter(buf, [off], val16, mask=mask)

    def chunk_body(c, _):
      scatter_chunk(c, ones16)
      pltpu.sync_copy(
          buf,
          out_hbm.at[pl.ds((batch0 + c * CHUNK_B) * BATCH_WORDS,
                           CHUNK_B * BATCH_WORDS)])
      scatter_chunk(c, zeros16)
      return 0

    lax.fori_loop(0, n_chunks, chunk_body, 0)

  return k(idx_flat)


def _tc_compute_body(idx_ref, out_ref):
  iota_d = lax.broadcasted_iota(jnp.int32, (TC_BLOCK_B, 26, DEPTH), 2)
  idx_b = idx_ref[...][:, :, None]
  out_ref[...] = jnp.where(idx_b == iota_d, jnp.float32(ON_VALUE),
                           jnp.float32(OFF_VALUE))


def _tc_assemble_body(img_ref, part_ref, out_ref, scratch, sem):
  del part_ref
  b = pl.program_id(0)
  cp = pltpu.make_async_copy(img_ref.at[pl.ds(b * 256, 256)], scratch, sem)
  cp.start()
  cp.wait()
  for g in range(4):
    rows = 8 if g < 3 else 2
    for c0 in range(8):
      cols = 128 if c0 < 7 else DEPTH - 7 * 128
      out_ref[0, pl.ds(8 * g, rows), pl.ds(128 * c0, cols)] = (
          scratch[pl.ds(8 * (8 * g + c0), rows), pl.ds(0, cols)])


@jax.jit
def kernel(indices):
  b_total, f_total = indices.shape
  n_tc_blocks = (b_total - SC_BATCHES) // TC_BLOCK_B
  sc_blocks = SC_BATCHES // TC_BLOCK_B

  img = _one_hot_sc_image(indices.reshape(-1), f_total)
  img2d = img.reshape(-1, 128)

  part = pl.pallas_call(
      _tc_compute_body,
      grid=(n_tc_blocks,),
      in_specs=[pl.BlockSpec((TC_BLOCK_B, f_total),
                             lambda i: (i + sc_blocks, 0))],
      out_specs=pl.BlockSpec((TC_BLOCK_B, f_total, DEPTH),
                             lambda i: (i + sc_blocks, 0, 0)),
      out_shape=jax.ShapeDtypeStruct((b_total, f_total, DEPTH), jnp.float32),
  )(indices)

  out = pl.pallas_call(
      _tc_assemble_body,
      grid=(SC_BATCHES,),
      in_specs=[
          pl.BlockSpec(memory_space=pl.ANY),
          pl.BlockSpec(memory_space=pl.ANY),
      ],
      out_specs=pl.BlockSpec((1, f_total, DEPTH), lambda b: (b, 0, 0)),
      out_shape=jax.ShapeDtypeStruct((b_total, f_total, DEPTH), jnp.float32),
      scratch_shapes=[pltpu.VMEM((256, 128), jnp.float32),
                      pltpu.SemaphoreType.DMA],
      input_output_aliases={1: 0},
  )(img2d, part)
  return out


# speedup vs baseline: 1.9253x; 1.9253x over previous
import functools

import jax
import jax.numpy as jnp
from jax import lax
from jax.experimental import pallas as pl
from jax.experimental.pallas import tpu as pltpu
from jax.experimental.pallas import tpu_sc as plsc

DEPTH = 1000
ON_VALUE = 1.0
OFF_VALUE = 0.0

NUM_CORES = 2
NUM_SUBCORES = 16
NUM_WORKERS = NUM_CORES * NUM_SUBCORES
LANES = 16

CHUNK_B = 2
F_PAD = 32
D_PAD = 1024
BATCH_WORDS = F_PAD * D_PAD

SC_BATCHES = 256
TC_BLOCK_B = 16


def _one_hot_sc_image(idx_flat, f_total):
  batches_per_worker = SC_BATCHES // NUM_WORKERS
  n_chunks = batches_per_worker // CHUNK_B
  chunk_rows = CHUNK_B * f_total
  rows_per_worker = batches_per_worker * f_total
  n_groups = -(-chunk_rows // LANES)

  mesh = plsc.VectorSubcoreMesh(core_axis_name="c", subcore_axis_name="s")

  @functools.partial(
      pl.kernel,
      mesh=mesh,
      out_type=jax.ShapeDtypeStruct((SC_BATCHES * BATCH_WORDS,), jnp.float32),
      scratch_types=[
          pltpu.VMEM((rows_per_worker,), jnp.int32),
          pltpu.VMEM((CHUNK_B * BATCH_WORDS,), jnp.float32),
      ],
      compiler_params=pltpu.CompilerParams(needs_layout_passes=False),
  )
  def k(idx_hbm, out_hbm, idx_v, buf):
    wid = lax.axis_index("s") * NUM_CORES + lax.axis_index("c")
    batch0 = wid * batches_per_worker

    pltpu.sync_copy(idx_hbm.at[pl.ds(batch0 * f_total, rows_per_worker)],
                    idx_v)

    zeros16 = jnp.zeros((LANES,), jnp.float32)

    def zero_body(i, _):
      base = i * (8 * LANES)
      for u in range(8):
        buf[pl.ds(base + u * LANES, LANES)] = zeros16
      return 0

    lax.fori_loop(0, CHUNK_B * BATCH_WORDS // (8 * LANES), zero_body, 0)

    lane = lax.iota(jnp.int32, LANES)
    ones16 = jnp.full((LANES,), jnp.float32(ON_VALUE))

    def scatter_chunk(c, val16):
      for g in range(n_groups):
        j = lane + g * LANES
        mask = j < chunk_rows if (g + 1) * LANES > chunk_rows else None
        d = plsc.load_gather(idx_v, [j + c * chunk_rows], mask=mask)
        b = jnp.where(j >= f_total, 1, 0)
        f = j - b * f_total
        off = (b * BATCH_WORDS + (f >> 3) * (8 * D_PAD) + (d >> 7) * 1024
               + (f & 7) * 128 + (d & 127))
        plsc.store_scatter(buf, [off], val16, mask=mask)

    def chunk_body(c, _):
      scatter_chunk(c, ones16)
      pltpu.sync_copy(
          buf,
          out_hbm.at[pl.ds((batch0 + c * CHUNK_B) * BATCH_WORDS,
                           CHUNK_B * BATCH_WORDS)])
      scatter_chunk(c, zeros16)
      return 0

    lax.fori_loop(0, n_chunks, chunk_body, 0)

  return k(idx_flat)


def _tc_compute_body(idx_ref, out_ref):
  iota_d = lax.broadcasted_iota(jnp.int32, (TC_BLOCK_B, 26, DEPTH), 2)
  idx_b = idx_ref[...][:, :, None]
  out_ref[...] = jnp.where(idx_b == iota_d, jnp.float32(ON_VALUE),
                           jnp.float32(OFF_VALUE))


def _tc_assemble_body(img_ref, part_ref, out_ref, scratch, sem):
  del part_ref
  i = pl.program_id(0)
  rows_per_block = TC_BLOCK_B * 256
  cp = pltpu.make_async_copy(
      img_ref.at[pl.ds(i * rows_per_block, rows_per_block)], scratch, sem)
  cp.start()
  cp.wait()
  for bb in range(TC_BLOCK_B):
    for g in range(4):
      rows = 8 if g < 3 else 2
      for c0 in range(8):
        cols = 128 if c0 < 7 else DEPTH - 7 * 128
        out_ref[bb, pl.ds(8 * g, rows), pl.ds(128 * c0, cols)] = (
            scratch[pl.ds(bb * 256 + 8 * (8 * g + c0), rows), pl.ds(0, cols)])


@jax.jit
def kernel(indices):
  b_total, f_total = indices.shape
  n_tc_blocks = (b_total - SC_BATCHES) // TC_BLOCK_B
  sc_blocks = SC_BATCHES // TC_BLOCK_B

  img = _one_hot_sc_image(indices.reshape(-1), f_total)
  img2d = img.reshape(-1, 128)

  part = pl.pallas_call(
      _tc_compute_body,
      grid=(n_tc_blocks,),
      in_specs=[pl.BlockSpec((TC_BLOCK_B, f_total),
                             lambda i: (i + sc_blocks, 0))],
      out_specs=pl.BlockSpec((TC_BLOCK_B, f_total, DEPTH),
                             lambda i: (i + sc_blocks, 0, 0)),
      out_shape=jax.ShapeDtypeStruct((b_total, f_total, DEPTH), jnp.float32),
  )(indices)

  out = pl.pallas_call(
      _tc_assemble_body,
      grid=(sc_blocks,),
      in_specs=[
          pl.BlockSpec(memory_space=pl.ANY),
          pl.BlockSpec(memory_space=pl.ANY),
      ],
      out_specs=pl.BlockSpec((TC_BLOCK_B, f_total, DEPTH),
                             lambda i: (i, 0, 0)),
      out_shape=jax.ShapeDtypeStruct((b_total, f_total, DEPTH), jnp.float32),
      scratch_shapes=[pltpu.VMEM((TC_BLOCK_B * 256, 128), jnp.float32),
                      pltpu.SemaphoreType.DMA],
      input_output_aliases={1: 0},
  )(img2d, part)
  return out
